# Initial kernel scaffold; baseline (speedup 1.0000x reference)
import jax
import jax.numpy as jnp
from jax.experimental import pallas as pl


def kernel(hidden_states, keys, values, reliability, Wq, Wdown, Wup):
    raise NotImplementedError("write your pallas kernel here")


# speedup vs baseline: 1.5205x; 1.5205x over previous
import functools
import math

import jax
import jax.numpy as jnp
from jax.experimental import pallas as pl
from jax.experimental.pallas import tpu as pltpu

_TOPK = 8
_TQ = 512
_TA = 512


def _qproj_kernel(hs_ref, wq_ref, q_ref, qsum_ref):
    q = jax.lax.dot_general(
        hs_ref[...], wq_ref[...], (((1,), (1,)), ((), ())),
        preferred_element_type=jnp.float32)
    q_ref[...] = q

    @pl.when(pl.program_id(0) == 0)
    def _init():
        qsum_ref[...] = jnp.zeros_like(qsum_ref)

    qsum_ref[...] += jnp.sum(q, axis=0, keepdims=True)


def _route_kernel(qsum_ref, keys_ref, rel_ref, values_ref, wdown_ref,
                  ksel_ref, bias_ref, vd_ref, vsel_ref, *, n_rows):
    nk = keys_ref.shape[0]
    scores = jax.lax.dot_general(
        keys_ref[...], qsum_ref[...], (((1,), (1,)), ((), ())),
        preferred_element_type=jnp.float32) * (1.0 / n_rows)
    scores = scores + jnp.log(jnp.clip(rel_ref[...], 1e-10))
    iota = jax.lax.broadcasted_iota(jnp.int32, (nk, 1), 0)
    for j in range(_TOPK):
        m = jnp.max(scores)
        idx = jnp.min(jnp.where(scores == m, iota, nk))
        ksel_ref[pl.ds(j, 1), :] = keys_ref[pl.ds(idx, 1), :]
        bias_ref[pl.ds(j, 1), :] = jnp.log(
            jnp.clip(rel_ref[pl.ds(idx, 1), :], 1e-10))
        vsel_ref[pl.ds(j, 1), :] = values_ref[pl.ds(idx, 1), :]
        scores = jnp.where(iota == idx, -jnp.inf, scores)
    vd_ref[...] = jax.lax.dot_general(
        vsel_ref[...], wdown_ref[...], (((1,), (1,)), ((), ())),
        preferred_element_type=jnp.float32)


def _attn_kernel(q_ref, ksel_ref, bias_ref, vd_ref, wup_ref, out_ref, *, scale):
    s = jax.lax.dot_general(
        q_ref[...], ksel_ref[...], (((1,), (1,)), ((), ())),
        preferred_element_type=jnp.float32) * scale + bias_ref[...]
    m = jnp.max(s, axis=-1, keepdims=True)
    e = jnp.exp(s - m)
    w = e / jnp.sum(e, axis=-1, keepdims=True)
    mid = jax.lax.dot_general(
        w, vd_ref[...], (((1,), (0,)), ((), ())),
        preferred_element_type=jnp.float32)
    g = mid * 0.5 * (1.0 + jax.lax.erf(mid * (1.0 / math.sqrt(2.0))))
    out_ref[...] = jax.lax.dot_general(
        g, wup_ref[...], (((1,), (1,)), ((), ())),
        preferred_element_type=jnp.float32)


def kernel(hidden_states, keys, values, reliability, Wq, Wdown, Wup):
    b, s, h = hidden_states.shape
    n = b * s
    nk, dk = keys.shape
    dv = Wdown.shape[0]
    hs = hidden_states.reshape(n, h)
    rel2 = reliability.reshape(nk, 1)

    q, qsum = pl.pallas_call(
        _qproj_kernel,
        grid=(n // _TQ,),
        in_specs=[
            pl.BlockSpec((_TQ, h), lambda i: (i, 0)),
            pl.BlockSpec((dk, h), lambda i: (0, 0)),
        ],
        out_specs=[
            pl.BlockSpec((_TQ, dk), lambda i: (i, 0)),
            pl.BlockSpec((1, dk), lambda i: (0, 0)),
        ],
        out_shape=[
            jax.ShapeDtypeStruct((n, dk), jnp.float32),
            jax.ShapeDtypeStruct((1, dk), jnp.float32),
        ],
    )(hs, Wq)

    ksel, bias8, vd = pl.pallas_call(
        functools.partial(_route_kernel, n_rows=n),
        in_specs=[
            pl.BlockSpec((1, dk), lambda: (0, 0)),
            pl.BlockSpec((nk, dk), lambda: (0, 0)),
            pl.BlockSpec((nk, 1), lambda: (0, 0)),
            pl.BlockSpec((nk, h), lambda: (0, 0)),
            pl.BlockSpec((dv, h), lambda: (0, 0)),
        ],
        out_specs=[
            pl.BlockSpec((_TOPK, dk), lambda: (0, 0)),
            pl.BlockSpec((_TOPK, 1), lambda: (0, 0)),
            pl.BlockSpec((_TOPK, dv), lambda: (0, 0)),
        ],
        out_shape=[
            jax.ShapeDtypeStruct((_TOPK, dk), jnp.float32),
            jax.ShapeDtypeStruct((_TOPK, 1), jnp.float32),
            jax.ShapeDtypeStruct((_TOPK, dv), jnp.float32),
        ],
        scratch_shapes=[pltpu.VMEM((_TOPK, h), jnp.float32)],
    )(qsum, keys, rel2, values, Wdown)

    out = pl.pallas_call(
        functools.partial(_attn_kernel, scale=1.0 / math.sqrt(dk)),
        grid=(n // _TA,),
        in_specs=[
            pl.BlockSpec((_TA, dk), lambda i: (i, 0)),
            pl.BlockSpec((_TOPK, dk), lambda i: (0, 0)),
            pl.BlockSpec((1, _TOPK), lambda i: (0, 0)),
            pl.BlockSpec((_TOPK, dv), lambda i: (0, 0)),
            pl.BlockSpec((h, dv), lambda i: (0, 0)),
        ],
        out_specs=pl.BlockSpec((_TA, h), lambda i: (i, 0)),
        out_shape=jax.ShapeDtypeStruct((n, h), jnp.float32),
    )(q, ksel, bias8.reshape(1, _TOPK), vd, Wup)

    return out.reshape(b, s, h)


# speedup vs baseline: 1.6715x; 1.0993x over previous
import functools
import math

import jax
import jax.numpy as jnp
from jax.experimental import pallas as pl
from jax.experimental.pallas import tpu as pltpu

_TOPK = 8
_T = 512


def _fused_kernel(hs_ref, wq_ref, keys_ref, rel_ref, wdown_ref, wup_ref,
                  values_hbm, out_ref,
                  q_scr, qsum_scr, ksel_scr, relc_scr, vd_scr, vsel_scr, sem,
                  *, n_rows, scale):
    p = pl.program_id(0)
    i = pl.program_id(1)
    nk = keys_ref.shape[0]

    @pl.when(p == 0)
    def _qproj():
        q = jax.lax.dot_general(
            hs_ref[...], wq_ref[...], (((1,), (1,)), ((), ())),
            preferred_element_type=jnp.float32)
        q_scr[pl.ds(i * _T, _T), :] = q

        @pl.when(i == 0)
        def _init():
            qsum_scr[...] = jnp.zeros_like(qsum_scr)

        qsum_scr[...] += jnp.sum(q, axis=0, keepdims=True)

    @pl.when((p == 1) & (i == 0))
    def _route():
        scores = jax.lax.dot_general(
            keys_ref[...], qsum_scr[...], (((1,), (1,)), ((), ())),
            preferred_element_type=jnp.float32) * (1.0 / n_rows)
        scores = scores + jnp.log(jnp.clip(rel_ref[...], 1e-10))
        iota = jax.lax.broadcasted_iota(jnp.int32, (nk, 1), 0)
        copies = []
        for j in range(_TOPK):
            m = jnp.max(scores)
            idx = jnp.min(jnp.where(scores == m, iota, nk))
            ksel_scr[pl.ds(j, 1), :] = keys_ref[pl.ds(idx, 1), :]
            relc_scr[pl.ds(j, 1), :] = jnp.clip(rel_ref[pl.ds(idx, 1), :],
                                                1e-10)
            cp = pltpu.make_async_copy(
                values_hbm.at[pl.ds(idx, 1), :], vsel_scr.at[pl.ds(j, 1), :],
                sem)
            cp.start()
            copies.append(cp)
            scores = jnp.where(iota == idx, -jnp.inf, scores)
        for cp in copies:
            cp.wait()
        vd = jax.lax.dot_general(
            vsel_scr[...], wdown_ref[...], (((1,), (1,)), ((), ())),
            preferred_element_type=jnp.float32)
        vd_scr[...] = vd * relc_scr[...]

    @pl.when(p == 1)
    def _attn():
        q = q_scr[pl.ds(i * _T, _T), :]
        s = jax.lax.dot_general(
            q, ksel_scr[...], (((1,), (1,)), ((), ())),
            preferred_element_type=jnp.float32) * scale
        m = jnp.max(s, axis=-1, keepdims=True)
        e = jnp.exp(s - m)
        denom = jax.lax.dot_general(
            e, relc_scr[...], (((1,), (0,)), ((), ())),
            preferred_element_type=jnp.float32)
        u = jax.lax.dot_general(
            e, vd_scr[...], (((1,), (0,)), ((), ())),
            preferred_element_type=jnp.float32)
        mid = u / denom
        g = mid * 0.5 * (1.0 + jax.lax.erf(mid * (1.0 / math.sqrt(2.0))))
        out_ref[...] = jax.lax.dot_general(
            g, wup_ref[...], (((1,), (1,)), ((), ())),
            preferred_element_type=jnp.float32)


def kernel(hidden_states, keys, values, reliability, Wq, Wdown, Wup):
    b, s, h = hidden_states.shape
    n = b * s
    nk, dk = keys.shape
    dv = Wdown.shape[0]
    hs = hidden_states.reshape(n, h)
    rel2 = reliability.reshape(nk, 1)
    nt = n // _T

    out = pl.pallas_call(
        functools.partial(_fused_kernel, n_rows=n, scale=1.0 / math.sqrt(dk)),
        grid=(2, nt),
        in_specs=[
            pl.BlockSpec((_T, h), lambda p, i: (jnp.where(p == 0, i, nt - 1), 0)),
            pl.BlockSpec((dk, h), lambda p, i: (0, 0)),
            pl.BlockSpec((nk, dk), lambda p, i: (0, 0)),
            pl.BlockSpec((nk, 1), lambda p, i: (0, 0)),
            pl.BlockSpec((dv, h), lambda p, i: (0, 0)),
            pl.BlockSpec((h, dv), lambda p, i: (0, 0)),
            pl.BlockSpec(memory_space=pl.ANY),
        ],
        out_specs=pl.BlockSpec((_T, h), lambda p, i: (jnp.where(p == 0, 0, i), 0)),
        out_shape=jax.ShapeDtypeStruct((n, h), jnp.float32),
        scratch_shapes=[
            pltpu.VMEM((n, dk), jnp.float32),
            pltpu.VMEM((1, dk), jnp.float32),
            pltpu.VMEM((_TOPK, dk), jnp.float32),
            pltpu.VMEM((_TOPK, 1), jnp.float32),
            pltpu.VMEM((_TOPK, dv), jnp.float32),
            pltpu.VMEM((_TOPK, h), jnp.float32),
            pltpu.SemaphoreType.DMA,
        ],
    )(hs, Wq, keys, rel2, Wdown, Wup, values)

    return out.reshape(b, s, h)


# speedup vs baseline: 1.7277x; 1.0336x over previous
import functools
import math

import jax
import jax.numpy as jnp
from jax.experimental import pallas as pl
from jax.experimental.pallas import tpu as pltpu

_TOPK = 8
_T = 512


def _fused_kernel(hs_ref, wq_ref, keys_ref, rel_ref, wdown_ref, wup_ref,
                  values_hbm, out_ref,
                  q_scr, qsum_scr, ksel_scr, relc_scr, vd_scr, vsel_scr, sem,
                  *, n_rows, scale):
    p = pl.program_id(0)
    i = pl.program_id(1)
    nk = keys_ref.shape[0]

    @pl.when(p == 0)
    def _qproj():
        q = jax.lax.dot_general(
            hs_ref[...], wq_ref[...], (((1,), (1,)), ((), ())),
            preferred_element_type=jnp.float32)
        q_scr[pl.ds(i * _T, _T), :] = q

        @pl.when(i == 0)
        def _init():
            qsum_scr[...] = jnp.zeros_like(qsum_scr)

        qsum_scr[...] += jnp.sum(q, axis=0, keepdims=True)

    @pl.when((p == 1) & (i == 0))
    def _route():
        scores = jax.lax.dot_general(
            qsum_scr[...], keys_ref[...], (((1,), (1,)), ((), ())),
            preferred_element_type=jnp.float32) * (1.0 / n_rows)
        scores = scores + jnp.log(jnp.clip(rel_ref[...], 1e-10))
        iota = jax.lax.broadcasted_iota(jnp.int32, (1, nk), 1)
        copies = []
        for j in range(_TOPK):
            m = jnp.max(scores)
            idx = jnp.min(jnp.where(scores == m, iota, nk))
            ksel_scr[pl.ds(j, 1), :] = keys_ref[pl.ds(idx, 1), :]
            rel_j = jnp.max(jnp.where(iota == idx, rel_ref[...], -1.0))
            relc_scr[pl.ds(j, 1), :] = jnp.full((1, 1), jnp.clip(rel_j, 1e-10),
                                                jnp.float32)
            cp = pltpu.make_async_copy(
                values_hbm.at[pl.ds(idx, 1), :], vsel_scr.at[pl.ds(j, 1), :],
                sem)
            cp.start()
            copies.append(cp)
            scores = jnp.where(iota == idx, -jnp.inf, scores)
        for cp in copies:
            cp.wait()
        vd = jax.lax.dot_general(
            vsel_scr[...], wdown_ref[...], (((1,), (1,)), ((), ())),
            preferred_element_type=jnp.float32)
        vd_scr[...] = vd * relc_scr[...]

    @pl.when(p == 1)
    def _attn():
        q = q_scr[pl.ds(i * _T, _T), :]
        s = jax.lax.dot_general(
            q, ksel_scr[...], (((1,), (1,)), ((), ())),
            preferred_element_type=jnp.float32) * scale
        m = jnp.max(s, axis=-1, keepdims=True)
        e = jnp.exp(s - m)
        denom = jax.lax.dot_general(
            e, relc_scr[...], (((1,), (0,)), ((), ())),
            preferred_element_type=jnp.float32)
        u = jax.lax.dot_general(
            e, vd_scr[...], (((1,), (0,)), ((), ())),
            preferred_element_type=jnp.float32)
        mid = u / denom
        g = mid * 0.5 * (1.0 + jax.lax.erf(mid * (1.0 / math.sqrt(2.0))))
        out_ref[...] = jax.lax.dot_general(
            g, wup_ref[...], (((1,), (1,)), ((), ())),
            preferred_element_type=jnp.float32)


def kernel(hidden_states, keys, values, reliability, Wq, Wdown, Wup):
    b, s, h = hidden_states.shape
    n = b * s
    nk, dk = keys.shape
    dv = Wdown.shape[0]
    hs = hidden_states.reshape(n, h)
    rel_row = reliability.reshape(1, nk)
    nt = n // _T

    out = pl.pallas_call(
        functools.partial(_fused_kernel, n_rows=n, scale=1.0 / math.sqrt(dk)),
        grid=(2, nt),
        in_specs=[
            pl.BlockSpec((_T, h), lambda p, i: (jnp.where(p == 0, i, nt - 1), 0)),
            pl.BlockSpec((dk, h), lambda p, i: (0, 0)),
            pl.BlockSpec((nk, dk), lambda p, i: (0, 0)),
            pl.BlockSpec((1, nk), lambda p, i: (0, 0)),
            pl.BlockSpec((dv, h), lambda p, i: (0, 0)),
            pl.BlockSpec((h, dv), lambda p, i: (0, 0)),
            pl.BlockSpec(memory_space=pl.ANY),
        ],
        out_specs=pl.BlockSpec((_T, h), lambda p, i: (jnp.where(p == 0, 0, i), 0)),
        out_shape=jax.ShapeDtypeStruct((n, h), jnp.float32),
        scratch_shapes=[
            pltpu.VMEM((n, dk), jnp.float32),
            pltpu.VMEM((1, dk), jnp.float32),
            pltpu.VMEM((_TOPK, dk), jnp.float32),
            pltpu.VMEM((_TOPK, 1), jnp.float32),
            pltpu.VMEM((_TOPK, dv), jnp.float32),
            pltpu.VMEM((_TOPK, h), jnp.float32),
            pltpu.SemaphoreType.DMA,
        ],
        compiler_params=pltpu.CompilerParams(
            vmem_limit_bytes=62 * 1024 * 1024),
    )(hs, Wq, keys, rel_row, Wdown, Wup, values)

    return out.reshape(b, s, h)
